# baseline (speedup 1.0000x reference)
import jax
import jax.numpy as jnp
from jax.experimental import pallas as pl
from jax.experimental.pallas import tpu as pltpu

B, S, D, E, K = 4, 2048, 2048, 16, 2
NOISY_STD = 1.0
T = 1024


def _gate_body(x_ref, wt_ref, nw_ref, noise_ref, w_out_ref, idx_out_ref):
    logits = jax.lax.dot_general(
        x_ref[...], wt_ref[...],
        (((1,), (0,)), ((), ())),
        preferred_element_type=jnp.float32,
    )
    logits = logits + noise_ref[...] * (NOISY_STD * nw_ref[...])

    iota = jax.lax.broadcasted_iota(jnp.int32, (T, E), 1)
    neg_inf = jnp.float32(-jnp.inf)

    m1 = jnp.max(logits, axis=1, keepdims=True)
    idx1 = jnp.min(jnp.where(logits == m1, iota, E), axis=1, keepdims=True)
    masked = jnp.where(iota == idx1, neg_inf, logits)
    m2 = jnp.max(masked, axis=1, keepdims=True)
    idx2 = jnp.min(jnp.where(masked == m2, iota, E), axis=1, keepdims=True)

    e2 = jnp.exp(m2 - m1)
    denom = 1.0 + e2
    w1 = 1.0 / denom
    w2 = e2 / denom

    w_out_ref[...] = jnp.where(
        iota == idx1, w1, jnp.where(iota == idx2, w2, jnp.float32(0.0)))
    idx_out_ref[...] = jnp.concatenate([idx1, idx2], axis=1)


@jax.jit
def kernel(x, W, noise_weight, noise):
    n = B * S
    x2 = x.reshape(n, D)
    wt = W.T
    nw = noise_weight.reshape(1, E)
    noise2 = noise.reshape(n, E)

    grid = (n // T,)
    weights, idx = pl.pallas_call(
        _gate_body,
        grid=grid,
        in_specs=[
            pl.BlockSpec((T, D), lambda i: (i, 0)),
            pl.BlockSpec((D, E), lambda i: (0, 0)),
            pl.BlockSpec((1, E), lambda i: (0, 0)),
            pl.BlockSpec((T, E), lambda i: (i, 0)),
        ],
        out_specs=[
            pl.BlockSpec((T, E), lambda i: (i, 0)),
            pl.BlockSpec((T, K), lambda i: (i, 0)),
        ],
        out_shape=[
            jax.ShapeDtypeStruct((n, E), jnp.float32),
            jax.ShapeDtypeStruct((n, K), jnp.int32),
        ],
        compiler_params=pltpu.CompilerParams(
            dimension_semantics=("arbitrary",),
        ),
    )(x2, wt, nw, noise2)

    return weights.reshape(B, S, E), idx.reshape(B, S, K)
